# baseline (speedup 1.0000x reference)
import functools

import jax
import jax.numpy as jnp
from jax import lax
from jax.experimental import pallas as pl
from jax.experimental.pallas import tpu as pltpu
from jax.experimental.pallas import tpu_sc as plsc

SUB = 128


@functools.cache
def _build_gather(V, D, B, K):
    mesh = plsc.VectorSubcoreMesh(core_axis_name="c", subcore_axis_name="s")
    NC, NS = mesh.num_cores, mesh.num_subcores
    NW = NC * NS
    CH = K * SUB
    b_per_w = B // NW
    n_steps = b_per_w // CH
    assert b_per_w % CH == 0

    @functools.partial(
        pl.kernel,
        out_type=jax.ShapeDtypeStruct((B, D), jnp.float32),
        mesh=mesh,
        scratch_types=[
            pltpu.VMEM((K, SUB), jnp.int32),
            pltpu.VMEM((CH, D), jnp.float32),
            pltpu.SemaphoreType.DMA,
        ],
        compiler_params=pltpu.CompilerParams(use_tc_tiling_on_sc=False),
    )
    def gather_kernel(x_hbm, table_hbm, out_hbm, idx_v, rows_v, sem):
        wid = lax.axis_index("s") * NC + lax.axis_index("c")
        base = wid * (b_per_w // SUB)

        def step(i, carry):
            row0 = base + i * K
            pltpu.sync_copy(x_hbm.at[pl.ds(row0, K)], idx_v)
            copies = [
                pltpu.async_copy(
                    table_hbm.at[idx_v.at[j]],
                    rows_v.at[pl.ds(j * SUB, SUB)],
                    sem,
                )
                for j in range(K)
            ]
            for c in copies:
                c.wait()
            pltpu.sync_copy(rows_v, out_hbm.at[pl.ds(row0 * SUB, CH)])
            return carry

        lax.fori_loop(0, n_steps, step, 0)

    return gather_kernel


def kernel(x, embedding):
    V, D = embedding.shape
    B = x.size
    x2d = x.reshape(B // SUB, SUB).astype(jnp.int32)
    out = _build_gather(V, D, B, 8)(x2d, embedding)
    return out.reshape(x.shape + (D,))


# speedup vs baseline: 1.0147x; 1.0147x over previous
import functools

import jax
import jax.numpy as jnp
from jax import lax
from jax.experimental import pallas as pl
from jax.experimental.pallas import tpu as pltpu
from jax.experimental.pallas import tpu_sc as plsc

SUB = 128


@functools.cache
def _build_gather(V, D, B, K):
    mesh = plsc.VectorSubcoreMesh(core_axis_name="c", subcore_axis_name="s")
    NC, NS = mesh.num_cores, mesh.num_subcores
    NW = NC * NS
    CH = K * SUB
    b_per_w = B // NW
    n_steps = b_per_w // CH
    assert b_per_w % CH == 0 and n_steps % 2 == 0 and n_steps >= 4

    @functools.partial(
        pl.kernel,
        out_type=jax.ShapeDtypeStruct((B, D), jnp.float32),
        mesh=mesh,
        scratch_types=[
            pltpu.VMEM((2, K, SUB), jnp.int32),
            pltpu.VMEM((2, CH, D), jnp.float32),
            pltpu.SemaphoreType.DMA,
            pltpu.SemaphoreType.DMA,
            pltpu.SemaphoreType.DMA,
            pltpu.SemaphoreType.DMA,
        ],
        compiler_params=pltpu.CompilerParams(use_tc_tiling_on_sc=False),
    )
    def gather_kernel(x_hbm, table_hbm, out_hbm, idx_v, rows_v,
                      gsem0, gsem1, osem0, osem1):
        gsem = (gsem0, gsem1)
        osem = (osem0, osem1)
        wid = lax.axis_index("s") * NC + lax.axis_index("c")
        base_rows = wid * (b_per_w // SUB)

        def load_and_fire(b, c):
            row0 = base_rows + c * K
            pltpu.sync_copy(x_hbm.at[pl.ds(row0, K)], idx_v.at[b])
            for j in range(K):
                pltpu.async_copy(
                    table_hbm.at[idx_v.at[b, j]],
                    rows_v.at[b, pl.ds(j * SUB, SUB)],
                    gsem[b],
                )

        def wait_gathers(b):
            pltpu.make_async_copy(
                out_hbm.at[pl.ds(0, CH)], rows_v.at[b], gsem[b]
            ).wait()

        def fire_write(b, c):
            row0 = base_rows + c * K
            pltpu.async_copy(
                rows_v.at[b], out_hbm.at[pl.ds(row0 * SUB, CH)], osem[b]
            )

        def wait_write(b):
            pltpu.make_async_copy(
                out_hbm.at[pl.ds(0, CH)], rows_v.at[b], osem[b]
            ).wait()

        load_and_fire(0, 0)
        load_and_fire(1, 1)

        def body(t, carry):
            c = 2 * t
            wait_gathers(0)
            fire_write(0, c)
            wait_gathers(1)
            fire_write(1, c + 1)
            wait_write(0)
            load_and_fire(0, c + 2)
            wait_write(1)
            load_and_fire(1, c + 3)
            return carry

        lax.fori_loop(0, n_steps // 2 - 1, body, 0)

        wait_gathers(0)
        fire_write(0, n_steps - 2)
        wait_gathers(1)
        fire_write(1, n_steps - 1)
        wait_write(0)
        wait_write(1)

    return gather_kernel


def kernel(x, embedding):
    V, D = embedding.shape
    B = x.size
    x2d = x.reshape(B // SUB, SUB).astype(jnp.int32)
    out = _build_gather(V, D, B, 5)(x2d, embedding)
    return out.reshape(x.shape + (D,))


# speedup vs baseline: 1.0147x; 1.0000x over previous
import functools

import jax
import jax.numpy as jnp
from jax import lax
from jax.experimental import pallas as pl
from jax.experimental.pallas import tpu as pltpu
from jax.experimental.pallas import tpu_sc as plsc

M = 4


@functools.cache
def _build_gather(V, D, NI, NJ):
    mesh = plsc.VectorSubcoreMesh(core_axis_name="c", subcore_axis_name="s")
    NC, NS = mesh.num_cores, mesh.num_subcores
    NW = NC * NS
    rows_per_w = NI // NW
    n_steps = rows_per_w // M
    splits = []
    o = 0
    while o < NJ:
        l = min(128, NJ - o)
        splits.append((o, l))
        o += l
    assert NI % NW == 0 and rows_per_w % M == 0 and n_steps >= 4

    @functools.partial(
        pl.kernel,
        out_type=jax.ShapeDtypeStruct((NI, NJ, D), jnp.float32),
        mesh=mesh,
        scratch_types=[
            pltpu.VMEM((2, M, NJ), jnp.int32),
            pltpu.VMEM((2, M, NJ, D), jnp.float32),
            pltpu.SemaphoreType.DMA,
            pltpu.SemaphoreType.DMA,
            pltpu.SemaphoreType.DMA,
            pltpu.SemaphoreType.DMA,
        ],
        compiler_params=pltpu.CompilerParams(use_tc_tiling_on_sc=False),
    )
    def gather_kernel(x_hbm, table_hbm, out_hbm, idx_v, rows_v,
                      gsem0, gsem1, osem0, osem1):
        gsem = (gsem0, gsem1)
        osem = (osem0, osem1)
        wid = lax.axis_index("s") * NC + lax.axis_index("c")
        base_i = wid * rows_per_w

        def load_and_fire(b, c):
            i0 = base_i + c * M
            pltpu.sync_copy(x_hbm.at[pl.ds(i0, M)], idx_v.at[b])
            for m in range(M):
                for o, l in splits:
                    pltpu.async_copy(
                        table_hbm.at[idx_v.at[b, m, pl.ds(o, l)]],
                        rows_v.at[b, m, pl.ds(o, l)],
                        gsem[b],
                    )

        def wait_gathers(b):
            pltpu.make_async_copy(
                out_hbm.at[pl.ds(0, M)], rows_v.at[b], gsem[b]
            ).wait()

        def fire_write(b, c):
            i0 = base_i + c * M
            pltpu.async_copy(
                rows_v.at[b], out_hbm.at[pl.ds(i0, M)], osem[b]
            )

        def wait_write(b):
            pltpu.make_async_copy(
                out_hbm.at[pl.ds(0, M)], rows_v.at[b], osem[b]
            ).wait()

        load_and_fire(0, 0)
        load_and_fire(1, 1)

        def body(t, carry):
            c = 2 * t
            wait_gathers(0)
            fire_write(0, c)
            wait_gathers(1)
            fire_write(1, c + 1)
            wait_write(0)
            load_and_fire(0, c + 2)
            wait_write(1)
            load_and_fire(1, c + 3)
            return carry

        n_pairs = (n_steps - 2) // 2
        lax.fori_loop(0, n_pairs, body, 0)
        n2 = 2 * n_pairs

        if n_steps % 2 == 0:
            wait_gathers(0)
            fire_write(0, n2)
            wait_gathers(1)
            fire_write(1, n2 + 1)
            wait_write(0)
            wait_write(1)
        else:
            wait_gathers(0)
            fire_write(0, n2)
            wait_gathers(1)
            fire_write(1, n2 + 1)
            wait_write(0)
            load_and_fire(0, n2 + 2)
            wait_gathers(0)
            fire_write(0, n2 + 2)
            wait_write(1)
            wait_write(0)

    return gather_kernel


def kernel(x, embedding):
    V, D = embedding.shape
    NI, NJ = x.shape
    t128 = lax.optimization_barrier(embedding.reshape(-1))
    table = t128.reshape(V, D)
    return _build_gather(V, D, NI, NJ)(x.astype(jnp.int32), table)


# speedup vs baseline: 1.0155x; 1.0007x over previous
import functools

import jax
import jax.numpy as jnp
from jax import lax
from jax.experimental import pallas as pl
from jax.experimental.pallas import tpu as pltpu
from jax.experimental.pallas import tpu_sc as plsc

M = 4


@functools.cache
def _build_gather(V, D, NI, NJ):
    mesh = plsc.VectorSubcoreMesh(core_axis_name="c", subcore_axis_name="s")
    NC, NS = mesh.num_cores, mesh.num_subcores
    NW = NC * NS
    rows_per_w = NI // NW
    n_steps = rows_per_w // M
    splits = []
    o = 0
    while o < NJ:
        l = min(128, NJ - o)
        splits.append((o, l))
        o += l
    assert NI % NW == 0 and rows_per_w % M == 0 and n_steps >= 4

    @functools.partial(
        pl.kernel,
        out_type=jax.ShapeDtypeStruct((NI, NJ, D), jnp.float32),
        mesh=mesh,
        scratch_types=[
            pltpu.VMEM((2, M, NJ), jnp.int32),
            pltpu.VMEM((2, M, NJ, D), jnp.float32),
            pltpu.SemaphoreType.DMA,
            pltpu.SemaphoreType.DMA,
            pltpu.SemaphoreType.DMA,
            pltpu.SemaphoreType.DMA,
        ],
        compiler_params=pltpu.CompilerParams(use_tc_tiling_on_sc=False),
    )
    def gather_kernel(x_hbm, table_hbm, out_hbm, idx_v, rows_v,
                      gsem0, gsem1, osem0, osem1):
        gsem = (gsem0, gsem1)
        osem = (osem0, osem1)
        wid = lax.axis_index("s") * NC + lax.axis_index("c")
        base_i = wid * rows_per_w

        def load_and_fire(b, c):
            i0 = base_i + c * M
            pltpu.sync_copy(x_hbm.at[pl.ds(i0, M)], idx_v.at[b])
            for m in range(M):
                for o, l in splits:
                    pltpu.async_copy(
                        table_hbm.at[idx_v.at[b, m, pl.ds(o, l)]],
                        rows_v.at[b, m, pl.ds(o, l)],
                        gsem[b],
                    )

        def wait_gathers(b):
            pltpu.make_async_copy(
                out_hbm.at[pl.ds(0, M)], rows_v.at[b], gsem[b]
            ).wait()

        def fire_write(b, c):
            i0 = base_i + c * M
            pltpu.async_copy(
                rows_v.at[b], out_hbm.at[pl.ds(i0, M)], osem[b]
            )

        def wait_write(b):
            pltpu.make_async_copy(
                out_hbm.at[pl.ds(0, M)], rows_v.at[b], osem[b]
            ).wait()

        load_and_fire(0, 0)
        load_and_fire(1, 1)

        def body(t, carry):
            c = 2 * t
            wait_gathers(0)
            fire_write(0, c)
            wait_gathers(1)
            fire_write(1, c + 1)
            wait_write(0)
            load_and_fire(0, c + 2)
            wait_write(1)
            load_and_fire(1, c + 3)
            return carry

        n_pairs = (n_steps - 2) // 2
        lax.fori_loop(0, n_pairs, body, 0)
        n2 = 2 * n_pairs

        if n_steps % 2 == 0:
            wait_gathers(0)
            fire_write(0, n2)
            wait_gathers(1)
            fire_write(1, n2 + 1)
            wait_write(0)
            wait_write(1)
        else:
            wait_gathers(0)
            fire_write(0, n2)
            wait_gathers(1)
            fire_write(1, n2 + 1)
            wait_write(0)
            load_and_fire(0, n2 + 2)
            wait_gathers(0)
            fire_write(0, n2 + 2)
            wait_write(1)
            wait_write(0)

    return gather_kernel


def kernel(x, embedding):
    V, D = embedding.shape
    NI, NJ = x.shape
    t128 = lax.optimization_barrier(embedding.reshape(V * D // 128, 128))
    table = t128.reshape(V, D)
    return _build_gather(V, D, NI, NJ)(x.astype(jnp.int32), table)
